# Initial kernel scaffold; baseline (speedup 1.0000x reference)
import jax
import jax.numpy as jnp
from jax.experimental import pallas as pl


def kernel(xyz2, xyz1):
    raise NotImplementedError("write your pallas kernel here")


# speedup vs baseline: 3.7526x; 3.7526x over previous
import functools

import jax
import jax.numpy as jnp
from jax.experimental import pallas as pl
from jax.experimental.pallas import tpu as pltpu

B = 8
N1 = 4096
N2 = 4096
K = 32
QT = 256


def _knn_kernel(x1t_ref, x2_ref, out_ref):
    q = x1t_ref[0]
    c = x2_ref[0]
    yy = jnp.sum(q * q, axis=1, keepdims=True)
    xx = jnp.sum(c * c, axis=0, keepdims=True)
    cross = jax.lax.dot_general(
        q, c, (((1,), (0,)), ((), ())),
        preferred_element_type=jnp.float32)
    dist = jnp.maximum(xx + yy - 2.0 * cross, 0.0)

    col = jax.lax.broadcasted_iota(jnp.int32, (QT, N2), 1)
    kk = jax.lax.broadcasted_iota(jnp.int32, (QT, K), 1)

    def body(k, carry):
        vals, out = carry
        m = jnp.min(vals, axis=1, keepdims=True)
        idx = jnp.min(jnp.where(vals == m, col, N2),
                      axis=1, keepdims=True)
        out = jnp.where(kk == k, idx, out)
        vals = jnp.where(col == idx, jnp.inf, vals)
        return vals, out

    _, out = jax.lax.fori_loop(
        0, K, body, (dist, jnp.zeros((QT, K), jnp.int32)))
    out_ref[0] = out


@functools.partial(jax.jit, static_argnames=())
def kernel(xyz2, xyz1):
    x1t = jnp.transpose(xyz1, (0, 2, 1))
    grid = (B, N1 // QT)
    out = pl.pallas_call(
        _knn_kernel,
        grid=grid,
        in_specs=[
            pl.BlockSpec((1, QT, 3), lambda b, i: (b, i, 0)),
            pl.BlockSpec((1, 3, N2), lambda b, i: (b, 0, 0)),
        ],
        out_specs=pl.BlockSpec((1, QT, K), lambda b, i: (b, i, 0)),
        out_shape=jax.ShapeDtypeStruct((B, N1, K), jnp.int32),
    )(x1t, xyz2)
    return out


# speedup vs baseline: 16.9814x; 4.5253x over previous
import functools

import jax
import jax.numpy as jnp
from jax.experimental import pallas as pl
from jax.experimental.pallas import tpu as pltpu

B = 8
N1 = 4096
N2 = 4096
K = 32
QT = 128
NG = 8
NJ = 64
NS = 8
NR = NG * NJ
R = 6
BIGF = float(2 ** 24)
INF = float("inf")


def _repmin(x):
    x = jnp.minimum(x, pltpu.roll(x, 4, 0))
    x = jnp.minimum(x, pltpu.roll(x, 2, 0))
    x = jnp.minimum(x, pltpu.roll(x, 1, 0))
    return x


def _knn_kernel(x2t_ref, x1_ref, rio_ref, out_ref):
    x2 = x2t_ref[0]
    q = x1_ref[0]
    yy = (q[0:1] * q[0:1] + q[1:2] * q[1:2]) + q[2:3] * q[2:3]
    xxcol = (x2[:, 0:1] * x2[:, 0:1] + x2[:, 1:2] * x2[:, 1:2]) \
        + x2[:, 2:3] * x2[:, 2:3]
    xxb = jnp.broadcast_to(xxcol, (N2, QT))
    cr = jax.lax.dot_general(
        x2, -2.0 * q, (((1,), (0,)), ((), ())),
        preferred_element_type=jnp.float32)
    dist = jnp.maximum((xxb + yy) + cr, 0.0)
    dist3 = dist.reshape(NR, NS, QT)
    rio = rio_ref[...].reshape(NR, NS, QT)

    vg = [dist3[g * NJ:(g + 1) * NJ] for g in range(NG)]
    rg = [rio[g * NJ:(g + 1) * NJ] for g in range(NG)]
    pvals, pidxs = [], []
    for r in range(R):
        for g in range(NG):
            v = vg[g]
            bmin = jnp.min(v, axis=0)
            bidx = jnp.min(jnp.where(v == bmin[None], rg[g], BIGF),
                           axis=0)
            pvals.append(bmin)
            pidxs.append(bidx)
            vg[g] = jnp.where(rg[g] == bidx[None], INF, v)

    pv = jnp.stack(pvals, axis=0)
    pi = jnp.stack(pidxs, axis=0)
    m = None
    for k in range(K):
        m = _repmin(jnp.min(pv, axis=0))
        cand = jnp.where(pv == m[None], pi, BIGF)
        idx = _repmin(jnp.min(cand, axis=0))
        out_ref[0, pl.ds(k, 1)] = idx[0:1].astype(jnp.int32)
        if k < K - 1:
            pv = jnp.where(pi == idx[None], INF, pv)
    m32 = m

    rem = vg[0]
    for g in range(1, NG):
        rem = jnp.minimum(rem, vg[g])
    rem = _repmin(jnp.min(rem, axis=0))
    bad = jnp.any(rem <= m32)

    @pl.when(bad)
    def _():
        def body(k, vals):
            fm = _repmin(jnp.min(vals, axis=0))
            cand = jnp.where(vals == fm[None], rio, BIGF)
            idx = _repmin(jnp.min(cand, axis=0))
            out_ref[0, pl.ds(k, 1)] = idx[0:1].astype(jnp.int32)
            return jnp.where(rio == idx[None], INF, vals)

        jax.lax.fori_loop(0, K, body, dist3)


@jax.jit
def kernel(xyz2, xyz1):
    x2t = jnp.transpose(xyz2, (0, 2, 1))
    rio = jnp.broadcast_to(
        jnp.arange(N2, dtype=jnp.float32)[:, None], (N2, QT))
    out = pl.pallas_call(
        _knn_kernel,
        grid=(B, N1 // QT),
        in_specs=[
            pl.BlockSpec((1, N2, 3), lambda b, i: (b, 0, 0)),
            pl.BlockSpec((1, 3, QT), lambda b, i: (b, 0, i)),
            pl.BlockSpec((N2, QT), lambda b, i: (0, 0)),
        ],
        out_specs=pl.BlockSpec((1, K, QT), lambda b, i: (b, 0, i)),
        out_shape=jax.ShapeDtypeStruct((B, K, N1), jnp.int32),
    )(x2t, xyz1, rio)
    return jnp.transpose(out, (0, 2, 1))


# speedup vs baseline: 21.5505x; 1.2691x over previous
import functools

import jax
import jax.numpy as jnp
from jax.experimental import pallas as pl
from jax.experimental.pallas import tpu as pltpu

B = 8
N1 = 4096
N2 = 4096
K = 32
QT = 256
NG = 8
NJ = 64
NS = 8
NR = NG * NJ
R = 6
BIGF = float(2 ** 24)
INF = float("inf")


def _repmin(x):
    x = jnp.minimum(x, pltpu.roll(x, 4, 0))
    x = jnp.minimum(x, pltpu.roll(x, 2, 0))
    x = jnp.minimum(x, pltpu.roll(x, 1, 0))
    return x


def _knn_kernel(x2t_ref, x1_ref, xx_ref, rio_ref, out_ref):
    x2 = x2t_ref[0]
    q = x1_ref[0]
    yy = (q[0:1] * q[0:1] + q[1:2] * q[1:2]) + q[2:3] * q[2:3]
    xxb = jnp.broadcast_to(xx_ref[0], (N2, QT))
    cr = jax.lax.dot_general(
        x2, -2.0 * q, (((1,), (0,)), ((), ())),
        preferred_element_type=jnp.float32)
    dist = jnp.maximum((xxb + yy) + cr, 0.0)
    dist3 = dist.reshape(NR, NS, QT)
    rio = rio_ref[...].reshape(NR, NS, QT)

    vg = [dist3[g * NJ:(g + 1) * NJ] for g in range(NG)]
    rg = [rio[g * NJ:(g + 1) * NJ] for g in range(NG)]
    pvals, pidxs = [], []
    for r in range(R):
        for g in range(NG):
            v = vg[g]
            bmin = jnp.min(v, axis=0)
            bidx = jnp.min(jnp.where(v == bmin[None], rg[g], BIGF),
                           axis=0)
            pvals.append(bmin)
            pidxs.append(bidx)
            vg[g] = jnp.where(rg[g] == bidx[None], INF, v)

    pv = jnp.stack(pvals, axis=0)
    pi = jnp.stack(pidxs, axis=0)
    m = None
    for k in range(K):
        m = _repmin(jnp.min(pv, axis=0))
        cand = jnp.where(pv == m[None], pi, BIGF)
        idx = _repmin(jnp.min(cand, axis=0))
        out_ref[0, pl.ds(k, 1)] = idx[0:1].astype(jnp.int32)
        if k < K - 1:
            pv = jnp.where(pi == idx[None], INF, pv)
    m32 = m

    rem = vg[0]
    for g in range(1, NG):
        rem = jnp.minimum(rem, vg[g])
    rem = _repmin(jnp.min(rem, axis=0))
    bad = jnp.any(rem <= m32)

    @pl.when(bad)
    def _():
        def body(k, vals):
            fm = _repmin(jnp.min(vals, axis=0))
            cand = jnp.where(vals == fm[None], rio, BIGF)
            idx = _repmin(jnp.min(cand, axis=0))
            out_ref[0, pl.ds(k, 1)] = idx[0:1].astype(jnp.int32)
            return jnp.where(rio == idx[None], INF, vals)

        jax.lax.fori_loop(0, K, body, dist3)


@jax.jit
def kernel(xyz2, xyz1):
    x2t = jnp.transpose(xyz2, (0, 2, 1))
    xx2 = jnp.sum(xyz2 ** 2, axis=1)[..., None]
    rio = jnp.broadcast_to(
        jnp.arange(N2, dtype=jnp.float32)[:, None], (N2, QT))
    out = pl.pallas_call(
        _knn_kernel,
        grid=(B, N1 // QT),
        in_specs=[
            pl.BlockSpec((1, N2, 3), lambda b, i: (b, 0, 0)),
            pl.BlockSpec((1, 3, QT), lambda b, i: (b, 0, i)),
            pl.BlockSpec((1, N2, 1), lambda b, i: (b, 0, 0)),
            pl.BlockSpec((N2, QT), lambda b, i: (0, 0)),
        ],
        out_specs=pl.BlockSpec((1, K, QT), lambda b, i: (b, 0, i)),
        out_shape=jax.ShapeDtypeStruct((B, K, N1), jnp.int32),
        compiler_params=pltpu.CompilerParams(
            dimension_semantics=("parallel", "parallel")),
    )(x2t, xyz1, xx2, rio)
    return jnp.transpose(out, (0, 2, 1))


# speedup vs baseline: 26.7082x; 1.2393x over previous
import functools

import jax
import jax.numpy as jnp
from jax.experimental import pallas as pl
from jax.experimental.pallas import tpu as pltpu

B = 8
N1 = 4096
N2 = 4096
K = 32
QT = 256
NG = 8
NJ = 64
NS = 8
NR = NG * NJ
R = 6
BIGF = float(2 ** 24)
INF = float("inf")


def _repmin(x):
    x = jnp.minimum(x, pltpu.roll(x, 4, 0))
    x = jnp.minimum(x, pltpu.roll(x, 2, 0))
    x = jnp.minimum(x, pltpu.roll(x, 1, 0))
    return x


def _knn_kernel(x2t_ref, x1_ref, xx_ref, rio_ref, out_ref):
    x2 = x2t_ref[0]
    q = x1_ref[0]
    yy = (q[0:1] * q[0:1] + q[1:2] * q[1:2]) + q[2:3] * q[2:3]
    xxb = jnp.broadcast_to(xx_ref[0], (N2, QT))
    cr = jax.lax.dot_general(
        x2, -2.0 * q, (((1,), (0,)), ((), ())),
        preferred_element_type=jnp.float32)
    dist = jnp.maximum((xxb + yy) + cr, 0.0)
    dist3 = dist.reshape(NR, NS, QT)
    rio = rio_ref[...].reshape(NR, NS, QT)

    vg = [dist3[g * NJ:(g + 1) * NJ] for g in range(NG)]
    rg = [rio[g * NJ:(g + 1) * NJ] for g in range(NG)]
    pvals, pidxs = [], []
    for r in range(R):
        for g in range(NG):
            v = vg[g]
            bmin = jnp.min(v, axis=0)
            bidx = jnp.min(jnp.where(v == bmin[None], rg[g], BIGF),
                           axis=0)
            pvals.append(bmin)
            pidxs.append(bidx)
            vg[g] = jnp.where(rg[g] == bidx[None], INF, v)

    pv = jnp.stack(pvals, axis=0)
    pi = jnp.stack(pidxs, axis=0)
    m = None
    for k in range(K):
        m = _repmin(jnp.min(pv, axis=0))
        cand = jnp.where(pv == m[None], pi, BIGF)
        idx = _repmin(jnp.min(cand, axis=0))
        out_ref[0, pl.ds(k, 1)] = idx[0:1].astype(jnp.int32)
        if k < K - 1:
            pv = jnp.where(pi == idx[None], INF, pv)
    m32 = m

    rem = vg[0]
    for g in range(1, NG):
        rem = jnp.minimum(rem, vg[g])
    rem = _repmin(jnp.min(rem, axis=0))
    bad = jnp.any(rem <= m32)

    @pl.when(bad)
    def _():
        def body(k, vals):
            fm = _repmin(jnp.min(vals, axis=0))
            cand = jnp.where(vals == fm[None], rio, BIGF)
            idx = _repmin(jnp.min(cand, axis=0))
            out_ref[0, pl.ds(k, 1)] = idx[0:1].astype(jnp.int32)
            return jnp.where(rio == idx[None], INF, vals)

        jax.lax.fori_loop(0, K, body, dist3)


def _knn_call(x2t, x1, xx2, rio):
    bsh = x2t.shape[0]
    return pl.pallas_call(
        _knn_kernel,
        grid=(bsh, N1 // QT),
        in_specs=[
            pl.BlockSpec((1, N2, 3), lambda b, i: (b, 0, 0)),
            pl.BlockSpec((1, 3, QT), lambda b, i: (b, 0, i)),
            pl.BlockSpec((1, N2, 1), lambda b, i: (b, 0, 0)),
            pl.BlockSpec((N2, QT), lambda b, i: (0, 0)),
        ],
        out_specs=pl.BlockSpec((1, K, QT), lambda b, i: (b, 0, i)),
        out_shape=jax.ShapeDtypeStruct((bsh, K, N1), jnp.int32),
        compiler_params=pltpu.CompilerParams(
            dimension_semantics=("parallel", "parallel")),
    )(x2t, x1, xx2, rio)


@jax.jit
def kernel(xyz2, xyz1):
    x2t = jnp.transpose(xyz2, (0, 2, 1))
    xx2 = jnp.sum(xyz2 ** 2, axis=1)[..., None]
    rio = jnp.broadcast_to(
        jnp.arange(N2, dtype=jnp.float32)[:, None], (N2, QT))
    ndev = len(jax.devices())
    nsh = 2 if (ndev >= 2 and B % 2 == 0) else 1
    if nsh > 1:
        mesh = jax.make_mesh((nsh,), ("d",))
        p = jax.sharding.PartitionSpec
        ns = lambda spec: jax.sharding.NamedSharding(mesh, spec)
        args = (jax.reshard(x2t, ns(p("d"))),
                jax.reshard(xyz1, ns(p("d"))),
                jax.reshard(xx2, ns(p("d"))),
                jax.reshard(rio, ns(p(None, None))))
        f = jax.shard_map(
            _knn_call, mesh=mesh,
            in_specs=(p("d"), p("d"), p("d"), p(None, None)),
            out_specs=p("d"), check_vma=False)
        out = f(*args)
    else:
        out = _knn_call(x2t, xyz1, xx2, rio)
    return jnp.transpose(out, (0, 2, 1))


# speedup vs baseline: 27.2172x; 1.0191x over previous
import functools

import jax
import jax.numpy as jnp
from jax.experimental import pallas as pl
from jax.experimental.pallas import tpu as pltpu

B = 8
N1 = 4096
N2 = 4096
K = 32
QT = 256
NG = 8
NJ = 64
NS = 8
NR = NG * NJ
R = 6
BIGF = float(2 ** 24)
INF = float("inf")


def _repmin(x):
    x = jnp.minimum(x, pltpu.roll(x, 4, 0))
    x = jnp.minimum(x, pltpu.roll(x, 2, 0))
    x = jnp.minimum(x, pltpu.roll(x, 1, 0))
    return x


def _knn_kernel(x2t_ref, x1_ref, xx_ref, rio_ref, out_ref):
    x2 = x2t_ref[0]
    q = x1_ref[0]
    yy = (q[0:1] * q[0:1] + q[1:2] * q[1:2]) + q[2:3] * q[2:3]
    xxb = jnp.broadcast_to(xx_ref[0], (N2, QT))
    cr = jax.lax.dot_general(
        x2, -2.0 * q, (((1,), (0,)), ((), ())),
        preferred_element_type=jnp.float32)
    dist = jnp.maximum((xxb + yy) + cr, 0.0)
    dist3 = dist.reshape(NR, NS, QT)
    rio = rio_ref[...].reshape(NR, NS, QT)

    vg = [dist3[g * NJ:(g + 1) * NJ] for g in range(NG)]
    rg = [rio[g * NJ:(g + 1) * NJ] for g in range(NG)]
    pvals, pidxs = [], []
    for r in range(R):
        for g in range(NG):
            v = vg[g]
            bmin = jnp.min(v, axis=0)
            bidx = jnp.min(jnp.where(v == bmin[None], rg[g], BIGF),
                           axis=0)
            pvals.append(bmin)
            pidxs.append(bidx)
            vg[g] = jnp.where(rg[g] == bidx[None], INF, v)

    pv = jnp.stack(pvals, axis=0)
    pi = jnp.stack(pidxs, axis=0)
    m = None
    for k in range(K):
        m = _repmin(jnp.min(pv, axis=0))
        cand = jnp.where(pv == m[None], pi, BIGF)
        idx = _repmin(jnp.min(cand, axis=0))
        out_ref[0, pl.ds(k, 1)] = idx[0:1].astype(jnp.int32)
        if k < K - 1:
            pv = jnp.where(pi == idx[None], INF, pv)
    m32 = m

    rem = vg[0]
    for g in range(1, NG):
        rem = jnp.minimum(rem, vg[g])
    rem = _repmin(jnp.min(rem, axis=0))
    bad = jnp.any(rem <= m32)

    @pl.when(bad)
    def _():
        def body(k, vals):
            fm = _repmin(jnp.min(vals, axis=0))
            cand = jnp.where(vals == fm[None], rio, BIGF)
            idx = _repmin(jnp.min(cand, axis=0))
            out_ref[0, pl.ds(k, 1)] = idx[0:1].astype(jnp.int32)
            return jnp.where(rio == idx[None], INF, vals)

        jax.lax.fori_loop(0, K, body, dist3)


def _knn_call(x2t, x1, xx2, rio):
    bsh = x2t.shape[0]
    return pl.pallas_call(
        _knn_kernel,
        grid=(bsh, N1 // QT),
        in_specs=[
            pl.BlockSpec((1, N2, 3), lambda b, i: (b, 0, 0)),
            pl.BlockSpec((1, 3, QT), lambda b, i: (b, 0, i)),
            pl.BlockSpec((1, N2, 1), lambda b, i: (b, 0, 0)),
            pl.BlockSpec((N2, QT), lambda b, i: (0, 0)),
        ],
        out_specs=pl.BlockSpec((1, K, QT), lambda b, i: (b, 0, i)),
        out_shape=jax.ShapeDtypeStruct((bsh, K, N1), jnp.int32),
        compiler_params=pltpu.CompilerParams(
            dimension_semantics=("parallel", "parallel")),
    )(x2t, x1, xx2, rio)


def _shard_fn(xyz2_sh, xyz1_sh):
    x2t = jnp.transpose(xyz2_sh, (0, 2, 1))
    xx2 = jnp.sum(xyz2_sh ** 2, axis=1)[..., None]
    rio = jnp.broadcast_to(
        jnp.arange(N2, dtype=jnp.float32)[:, None], (N2, QT))
    out = _knn_call(x2t, xyz1_sh, xx2, rio)
    return jnp.transpose(out, (0, 2, 1))


@jax.jit
def kernel(xyz2, xyz1):
    ndev = len(jax.devices())
    nsh = 2 if (ndev >= 2 and B % 2 == 0) else 1
    if nsh > 1:
        mesh = jax.make_mesh((nsh,), ("d",))
        p = jax.sharding.PartitionSpec
        ns = jax.sharding.NamedSharding(mesh, p("d"))
        f = jax.shard_map(
            _shard_fn, mesh=mesh,
            in_specs=(p("d"), p("d")),
            out_specs=p("d"), check_vma=False)
        return f(jax.reshard(xyz2, ns), jax.reshard(xyz1, ns))
    return _shard_fn(xyz2, xyz1)


# speedup vs baseline: 29.3327x; 1.0777x over previous
import functools

import jax
import jax.numpy as jnp
from jax.experimental import pallas as pl
from jax.experimental.pallas import tpu as pltpu

B = 8
N1 = 4096
N2 = 4096
K = 32
QT = 512
NG = 8
NJ = 64
NS = 8
NR = NG * NJ
R = 6
BIGF = float(2 ** 24)
INF = float("inf")


def _repmin(x):
    x = jnp.minimum(x, pltpu.roll(x, 4, 0))
    x = jnp.minimum(x, pltpu.roll(x, 2, 0))
    x = jnp.minimum(x, pltpu.roll(x, 1, 0))
    return x


def _knn_kernel(x2t_ref, x1_ref, xx_ref, rio_ref, out_ref):
    x2 = x2t_ref[0]
    q = x1_ref[0]
    yy = (q[0:1] * q[0:1] + q[1:2] * q[1:2]) + q[2:3] * q[2:3]
    xxb = jnp.broadcast_to(xx_ref[0], (N2, QT))
    cr = jax.lax.dot_general(
        x2, -2.0 * q, (((1,), (0,)), ((), ())),
        preferred_element_type=jnp.float32)
    dist = jnp.maximum((xxb + yy) + cr, 0.0)
    dist3 = dist.reshape(NR, NS, QT)
    rio = rio_ref[...].reshape(NR, NS, QT)

    vg = [dist3[g * NJ:(g + 1) * NJ] for g in range(NG)]
    rg = [rio[g * NJ:(g + 1) * NJ] for g in range(NG)]
    pvals, pidxs = [], []
    for r in range(R):
        for g in range(NG):
            v = vg[g]
            bmin = jnp.min(v, axis=0)
            bidx = jnp.min(jnp.where(v == bmin[None], rg[g], BIGF),
                           axis=0)
            pvals.append(bmin)
            pidxs.append(bidx)
            vg[g] = jnp.where(rg[g] == bidx[None], INF, v)

    pv = jnp.stack(pvals, axis=0)
    pi = jnp.stack(pidxs, axis=0)
    m = None
    for k in range(K):
        m = _repmin(jnp.min(pv, axis=0))
        cand = jnp.where(pv == m[None], pi, BIGF)
        idx = _repmin(jnp.min(cand, axis=0))
        out_ref[0, pl.ds(k, 1)] = idx[0:1].astype(jnp.int32)
        if k < K - 1:
            pv = jnp.where(pi == idx[None], INF, pv)
    m32 = m

    rem = vg[0]
    for g in range(1, NG):
        rem = jnp.minimum(rem, vg[g])
    rem = _repmin(jnp.min(rem, axis=0))
    bad = jnp.any(rem <= m32)

    @pl.when(bad)
    def _():
        def body(k, vals):
            fm = _repmin(jnp.min(vals, axis=0))
            cand = jnp.where(vals == fm[None], rio, BIGF)
            idx = _repmin(jnp.min(cand, axis=0))
            out_ref[0, pl.ds(k, 1)] = idx[0:1].astype(jnp.int32)
            return jnp.where(rio == idx[None], INF, vals)

        jax.lax.fori_loop(0, K, body, dist3)


def _knn_call(x2t, x1, xx2, rio):
    bsh = x2t.shape[0]
    return pl.pallas_call(
        _knn_kernel,
        grid=(bsh, N1 // QT),
        in_specs=[
            pl.BlockSpec((1, N2, 3), lambda b, i: (b, 0, 0)),
            pl.BlockSpec((1, 3, QT), lambda b, i: (b, 0, i)),
            pl.BlockSpec((1, N2, 1), lambda b, i: (b, 0, 0)),
            pl.BlockSpec((N2, QT), lambda b, i: (0, 0)),
        ],
        out_specs=pl.BlockSpec((1, K, QT), lambda b, i: (b, 0, i)),
        out_shape=jax.ShapeDtypeStruct((bsh, K, N1), jnp.int32),
        compiler_params=pltpu.CompilerParams(
            dimension_semantics=("parallel", "parallel")),
    )(x2t, x1, xx2, rio)


def _shard_fn(xyz2_sh, xyz1_sh):
    x2t = jnp.transpose(xyz2_sh, (0, 2, 1))
    xx2 = jnp.sum(xyz2_sh ** 2, axis=1)[..., None]
    rio = jnp.broadcast_to(
        jnp.arange(N2, dtype=jnp.float32)[:, None], (N2, QT))
    out = _knn_call(x2t, xyz1_sh, xx2, rio)
    return jnp.transpose(out, (0, 2, 1))


@jax.jit
def kernel(xyz2, xyz1):
    ndev = len(jax.devices())
    nsh = 2 if (ndev >= 2 and B % 2 == 0) else 1
    if nsh > 1:
        mesh = jax.make_mesh((nsh,), ("d",))
        p = jax.sharding.PartitionSpec
        ns = jax.sharding.NamedSharding(mesh, p("d"))
        f = jax.shard_map(
            _shard_fn, mesh=mesh,
            in_specs=(p("d"), p("d")),
            out_specs=p("d"), check_vma=False)
        return f(jax.reshard(xyz2, ns), jax.reshard(xyz1, ns))
    return _shard_fn(xyz2, xyz1)


# speedup vs baseline: 36.9154x; 1.2585x over previous
import functools

import jax
import jax.numpy as jnp
from jax.experimental import pallas as pl
from jax.experimental.pallas import tpu as pltpu

B = 8
N1 = 4096
N2 = 4096
K = 32
QT = 512
NG = 8
NJ = 64
NS = 8
NR = NG * NJ
R = 6
BIGF = float(2 ** 24)
INF = float("inf")


def _repmin(x):
    x = jnp.minimum(x, pltpu.roll(x, 4, 0))
    x = jnp.minimum(x, pltpu.roll(x, 2, 0))
    x = jnp.minimum(x, pltpu.roll(x, 1, 0))
    return x


def _knn_kernel(x2t_ref, x1_ref, xx_ref, rio_ref, out_ref):
    x2 = x2t_ref[0]
    q = x1_ref[0]
    yy = (q[0:1] * q[0:1] + q[1:2] * q[1:2]) + q[2:3] * q[2:3]
    xxb = jnp.broadcast_to(xx_ref[0], (N2, QT))
    cr = jax.lax.dot_general(
        x2, -2.0 * q, (((1,), (0,)), ((), ())),
        preferred_element_type=jnp.float32)
    dist = jnp.maximum((xxb + yy) + cr, 0.0)
    dist3 = dist.reshape(NR, NS, QT)
    rio = rio_ref[...].reshape(NR, NS, QT)

    vg = [dist3[g * NJ:(g + 1) * NJ] for g in range(NG)]
    rg = [rio[g * NJ:(g + 1) * NJ] for g in range(NG)]
    pvals, pidxs = [], []
    for r in range(R):
        for g in range(NG):
            v = vg[g]
            bmin = jnp.min(v, axis=0)
            bidx = jnp.min(jnp.where(v == bmin[None], rg[g], BIGF),
                           axis=0)
            pvals.append(bmin)
            pidxs.append(bidx)
            vg[g] = jnp.where(rg[g] == bidx[None], INF, v)

    pv = jnp.stack(pvals, axis=0)
    pi = jnp.stack(pidxs, axis=0)
    m = None
    for k in range(K):
        m = _repmin(jnp.min(pv, axis=0))
        cand = jnp.where(pv == m[None], pi, BIGF)
        idx = _repmin(jnp.min(cand, axis=0))
        out_ref[0, pl.ds(k, 1)] = idx[0:1].astype(jnp.int32)
        if k < K - 1:
            pv = jnp.where(pi == idx[None], INF, pv)
    m32 = m

    rem = vg[0]
    for g in range(1, NG):
        rem = jnp.minimum(rem, vg[g])
    rem = _repmin(jnp.min(rem, axis=0))
    bad = jnp.any(rem <= m32)

    @pl.when(bad)
    def _():
        def body(k, vals):
            fm = _repmin(jnp.min(vals, axis=0))
            cand = jnp.where(vals == fm[None], rio, BIGF)
            idx = _repmin(jnp.min(cand, axis=0))
            out_ref[0, pl.ds(k, 1)] = idx[0:1].astype(jnp.int32)
            return jnp.where(rio == idx[None], INF, vals)

        jax.lax.fori_loop(0, K, body, dist3)


def _knn_call(x2t, x1, xx2, rio):
    bsh = x2t.shape[0]
    return pl.pallas_call(
        _knn_kernel,
        grid=(bsh, N1 // QT),
        in_specs=[
            pl.BlockSpec((1, N2, 3), lambda b, i: (b, 0, 0)),
            pl.BlockSpec((1, 3, QT), lambda b, i: (b, 0, i)),
            pl.BlockSpec((1, N2, 1), lambda b, i: (b, 0, 0)),
            pl.BlockSpec((N2, QT), lambda b, i: (0, 0)),
        ],
        out_specs=pl.BlockSpec((1, K, QT), lambda b, i: (b, 0, i)),
        out_shape=jax.ShapeDtypeStruct((bsh, K, N1), jnp.int32),
        compiler_params=pltpu.CompilerParams(
            dimension_semantics=("parallel", "parallel")),
    )(x2t, x1, xx2, rio)


def _shard_fn(xyz2_sh, xyz1_sh):
    x2t = jnp.transpose(xyz2_sh, (0, 2, 1))
    xx2 = jnp.sum(xyz2_sh ** 2, axis=1)[..., None]
    rio = jnp.broadcast_to(
        jnp.arange(N2, dtype=jnp.float32)[:, None], (N2, QT))
    out = _knn_call(x2t, xyz1_sh, xx2, rio)
    return jnp.transpose(out, (0, 2, 1))


@jax.jit
def kernel(xyz2, xyz1):
    ndev = len(jax.devices())
    nsh = 2 if (ndev >= 2 and B % 2 == 0) else 1
    if nsh > 1:
        mesh = jax.make_mesh((nsh,), ("d",))
        p = jax.sharding.PartitionSpec
        nsrep = jax.sharding.NamedSharding(mesh, p())
        bsh = B // nsh

        def _sliced(xyz2_rep, xyz1_rep):
            st = jax.lax.axis_index("d") * bsh
            return _shard_fn(
                jax.lax.dynamic_slice_in_dim(xyz2_rep, st, bsh, 0),
                jax.lax.dynamic_slice_in_dim(xyz1_rep, st, bsh, 0))

        f = jax.shard_map(
            _sliced, mesh=mesh,
            in_specs=(p(), p()),
            out_specs=p("d"), check_vma=False)
        return f(jax.reshard(xyz2, nsrep), jax.reshard(xyz1, nsrep))
    return _shard_fn(xyz2, xyz1)
